# baseline (speedup 1.0000x reference)
import jax
import jax.numpy as jnp
from jax.experimental import pallas as pl
from jax.experimental.pallas import tpu as pltpu

_H = 28
_W = 28
_P = _H * _W
_B = 1024
_CHUNK = 256
_N_CHUNKS = _B // _CHUNK


def _axs_kernel(pos_ref, w_ref, x_hbm, out_hbm, a_ref, x_ref, o_ref,
                in_sems, out_sems):
    for c in range(_N_CHUNKS):
        pltpu.make_async_copy(
            x_hbm.at[pl.ds(c * _CHUNK, _CHUNK), :],
            x_ref.at[pl.ds(c * _CHUNK, _CHUNK), :],
            in_sems.at[c],
        ).start()

    def _flatten_rows(m):
        return jnp.concatenate([m[i:i + 1, :] for i in range(_H)], axis=1)

    pos0 = _flatten_rows(pos_ref[:, :, 0])
    pos1 = _flatten_rows(pos_ref[:, :, 1])
    sw = jnp.maximum(_flatten_rows(w_ref[:, :]), 0.0)
    r0 = jnp.round(pos0)
    r1 = jnp.round(pos1)
    q = jax.lax.broadcasted_iota(jnp.int32, (_P, _P), 0)
    u = (q // _W).astype(jnp.float32)
    v = (q % _W).astype(jnp.float32)
    d0 = u - pos0
    d1 = v - pos1
    inside = (jnp.abs(u - r0) < 2.5) & (jnp.abs(v - r1) < 2.5)
    a_ref[:, :] = jnp.where(
        inside, sw * jnp.exp(-0.5 * (d0 * d0 + d1 * d1)), 0.0
    )

    for c in range(_N_CHUNKS):
        sl = pl.ds(c * _CHUNK, _CHUNK)
        pltpu.make_async_copy(
            x_hbm.at[sl, :], x_ref.at[sl, :], in_sems.at[c]
        ).wait()
        o_ref[sl, :] = jnp.dot(
            x_ref[sl, :], a_ref[:, :],
            preferred_element_type=jnp.float32,
            precision=jax.lax.Precision.DEFAULT,
        )
        pltpu.make_async_copy(
            o_ref.at[sl, :], out_hbm.at[sl, :], out_sems.at[c]
        ).start()

    for c in range(_N_CHUNKS):
        sl = pl.ds(c * _CHUNK, _CHUNK)
        pltpu.make_async_copy(
            o_ref.at[sl, :], out_hbm.at[sl, :], out_sems.at[c]
        ).wait()


def kernel(input, pos2d, weight):
    x = input.reshape(_B, _P)

    out = pl.pallas_call(
        _axs_kernel,
        in_specs=[
            pl.BlockSpec(memory_space=pltpu.MemorySpace.VMEM),
            pl.BlockSpec(memory_space=pltpu.MemorySpace.VMEM),
            pl.BlockSpec(memory_space=pltpu.MemorySpace.HBM),
        ],
        out_specs=pl.BlockSpec(memory_space=pltpu.MemorySpace.HBM),
        out_shape=jax.ShapeDtypeStruct((_B, _P), jnp.float32),
        scratch_shapes=[
            pltpu.VMEM((_P, _P), jnp.float32),
            pltpu.VMEM((_B, _P), jnp.float32),
            pltpu.VMEM((_B, _P), jnp.float32),
            pltpu.SemaphoreType.DMA((_N_CHUNKS,)),
            pltpu.SemaphoreType.DMA((_N_CHUNKS,)),
        ],
    )(pos2d, weight, x)
    return out.reshape(input.shape)
